# baseline (speedup 1.0000x reference)
import functools

import jax
import jax.numpy as jnp
from jax import lax
from jax.experimental import pallas as pl
from jax.experimental.pallas import tpu as pltpu
from jax.experimental.pallas import tpu_sc as plsc

NUM_BINS = 32
EMBED_DIM = 64
SG = 4
L = 16
REP = 4


@functools.lru_cache(maxsize=None)
def _sc_lookup(nb: int, nseq: int):
    info = plsc.get_sparse_core_info()
    nc, ns = info.num_cores, info.num_subcores
    nw = nc * ns
    bat_w = nb // nw
    per_w = bat_w * nseq
    assert bat_w * nw == nb and bat_w == 2 * EMBED_DIM and nseq % (2 * SG) == 0
    n_groups = nseq // SG
    n_tiles_k = EMBED_DIM // 8
    mesh = plsc.VectorSubcoreMesh(core_axis_name="c", subcore_axis_name="s")

    scratch = [
        pltpu.VMEM((per_w,), jnp.int32),
        pltpu.VMEM((NUM_BINS, REP * EMBED_DIM), jnp.float32),
        pltpu.VMEM((SG * EMBED_DIM, bat_w), jnp.float32),
        pltpu.VMEM((SG * EMBED_DIM, bat_w), jnp.float32),
        pltpu.SemaphoreType.DMA,
        pltpu.SemaphoreType.DMA,
    ]

    @functools.partial(
        pl.kernel,
        out_type=jax.ShapeDtypeStruct((nseq * n_tiles_k * nw, 8, bat_w),
                                      jnp.float32),
        mesh=mesh,
        scratch_types=scratch,
        compiler_params=pltpu.CompilerParams(
            use_tc_tiling_on_sc=False, needs_layout_passes=False),
    )
    def k(idx_hbm, table_hbm, out_hbm, idx_v, table_v, rows0, rows1, s0, s1):
        wid = lax.axis_index("s") * nc + lax.axis_index("c")
        base = wid * per_w
        pltpu.sync_copy(table_hbm, table_v)
        pltpu.sync_copy(idx_hbm.at[pl.ds(base, per_w)], idx_v)

        lane = lax.iota(jnp.int32, L)
        lane_s = lane * nseq
        colmod = [(lane + d) & (L - 1) for d in range(L)]

        def compute_group(g, rows_ref):
            s_base = g * SG

            @plsc.parallel_loop(0, SG * (bat_w // L))
            def blk(t):
                s_loc = t // (bat_w // L)
                jb = t - s_loc * (bat_w // L)
                s = s_base + s_loc
                bins = plsc.load_gather(idx_v, [lane_s + (jb * (L * nseq) + s)])
                jv = lane + jb * L
                for cg in range(0, EMBED_DIM, L):
                    scal = s_loc * EMBED_DIM + cg
                    for d0 in range(0, L, 8):
                        colvs = [colmod[d0 + d] + scal for d in range(8)]
                        vs = [plsc.load_gather(table_v, [bins, cv])
                              for cv in colvs]
                        for cv, v in zip(colvs, vs):
                            plsc.store_scatter(rows_ref, [cv, jv], v)

        def store_descs(g, rows_ref, sem):
            descs = []
            for s_loc in range(SG):
                t_off = (g * SG + s_loc) * n_tiles_k * nw
                for kk in range(n_tiles_k):
                    descs.append(pltpu.make_async_copy(
                        rows_ref.at[pl.ds(s_loc * EMBED_DIM + kk * 8, 8)],
                        out_hbm.at[t_off + kk * nw + wid],
                        sem))
            return descs

        def fire_store(g, rows_ref, sem):
            for d in store_descs(g, rows_ref, sem):
                d.start()

        def wait_store(g, rows_ref, sem):
            for d in store_descs(g, rows_ref, sem):
                d.wait()

        compute_group(0, rows0)
        fire_store(0, rows0, s0)
        compute_group(1, rows1)
        fire_store(1, rows1, s1)

        def body(gh, carry):
            g0 = gh * 2
            wait_store(g0 - 2, rows0, s0)
            compute_group(g0, rows0)
            fire_store(g0, rows0, s0)
            wait_store(g0 - 1, rows1, s1)
            compute_group(g0 + 1, rows1)
            fire_store(g0 + 1, rows1, s1)
            return carry

        lax.fori_loop(1, n_groups // 2, body, 0)
        wait_store(n_groups - 2, rows0, s0)
        wait_store(n_groups - 1, rows1, s1)

    return k


def kernel(velocity_bins, table):
    b, s = velocity_bins.shape
    idx = velocity_bins.astype(jnp.int32).reshape(b * s)
    table_rep = jnp.tile(table, (1, REP))
    out3 = _sc_lookup(b, s)(idx, table_rep)
    out5 = out3.reshape(s, EMBED_DIM // 8, b // 128, 8, 128)
    return out5.transpose(2, 4, 0, 1, 3).reshape(b, s, EMBED_DIM)


# speedup vs baseline: 1.0387x; 1.0387x over previous
import functools

import jax
import jax.numpy as jnp
from jax import lax
from jax.experimental import pallas as pl
from jax.experimental.pallas import tpu as pltpu
from jax.experimental.pallas import tpu_sc as plsc

NUM_BINS = 32
EMBED_DIM = 64
SG = 4
L = 16
REP = 4


@functools.lru_cache(maxsize=None)
def _sc_lookup(nb: int, nseq: int):
    info = plsc.get_sparse_core_info()
    nc, ns = info.num_cores, info.num_subcores
    nw = nc * ns
    bat_w = nb // nw
    per_w = bat_w * nseq
    assert bat_w * nw == nb and bat_w == 2 * EMBED_DIM and nseq % (2 * SG) == 0
    n_groups = nseq // SG
    n_tiles_k = EMBED_DIM // 8
    mesh = plsc.VectorSubcoreMesh(core_axis_name="c", subcore_axis_name="s")

    scratch = [
        pltpu.VMEM((per_w,), jnp.int32),
        pltpu.VMEM((NUM_BINS, REP * EMBED_DIM), jnp.float32),
        pltpu.VMEM((SG * EMBED_DIM, bat_w), jnp.float32),
        pltpu.VMEM((SG * EMBED_DIM, bat_w), jnp.float32),
        pltpu.SemaphoreType.DMA,
        pltpu.SemaphoreType.DMA,
    ]

    @functools.partial(
        pl.kernel,
        out_type=jax.ShapeDtypeStruct((nseq * n_tiles_k * nw, 8, bat_w),
                                      jnp.float32),
        mesh=mesh,
        scratch_types=scratch,
        compiler_params=pltpu.CompilerParams(
            use_tc_tiling_on_sc=False, needs_layout_passes=False),
    )
    def k(idx_hbm, table_hbm, out_hbm, idx_v, table_v, rows0, rows1, s0, s1):
        wid = lax.axis_index("s") * nc + lax.axis_index("c")
        base = wid * per_w
        pltpu.sync_copy(table_hbm, table_v)
        pltpu.sync_copy(idx_hbm.at[pl.ds(base, per_w)], idx_v)

        lane = lax.iota(jnp.int32, L)
        lane_s = lane * nseq
        colmod = [(lane + d) & (L - 1) for d in range(L)]

        def compute_group(g, rows_ref):
            s_base = g * SG

            @plsc.parallel_loop(0, SG * (bat_w // L))
            def blk(t):
                s_loc = t // (bat_w // L)
                jb = t - s_loc * (bat_w // L)
                s = s_base + s_loc
                bins = plsc.load_gather(idx_v, [lane_s + (jb * (L * nseq) + s)])
                jv = lane + jb * L
                for cg in range(0, EMBED_DIM, L):
                    scal = s_loc * EMBED_DIM + cg
                    for d in range(L):
                        colv = colmod[d] + scal
                        v = plsc.load_gather(table_v, [bins, colv])
                        plsc.store_scatter(rows_ref, [colv, jv], v)

        def store_descs(g, rows_ref, sem):
            descs = []
            for s_loc in range(SG):
                t_off = (g * SG + s_loc) * n_tiles_k * nw
                for kk in range(n_tiles_k):
                    descs.append(pltpu.make_async_copy(
                        rows_ref.at[pl.ds(s_loc * EMBED_DIM + kk * 8, 8)],
                        out_hbm.at[t_off + kk * nw + wid],
                        sem))
            return descs

        def fire_store(g, rows_ref, sem):
            for d in store_descs(g, rows_ref, sem):
                d.start()

        def wait_store(g, rows_ref, sem):
            for d in store_descs(g, rows_ref, sem):
                d.wait()

        compute_group(0, rows0)
        fire_store(0, rows0, s0)
        compute_group(1, rows1)
        fire_store(1, rows1, s1)

        def body(gh, carry):
            g0 = gh * 2
            wait_store(g0 - 2, rows0, s0)
            compute_group(g0, rows0)
            fire_store(g0, rows0, s0)
            wait_store(g0 - 1, rows1, s1)
            compute_group(g0 + 1, rows1)
            fire_store(g0 + 1, rows1, s1)
            return carry

        lax.fori_loop(1, n_groups // 2, body, 0)
        wait_store(n_groups - 2, rows0, s0)
        wait_store(n_groups - 1, rows1, s1)

    return k


def kernel(velocity_bins, table):
    b, s = velocity_bins.shape
    idx = velocity_bins.astype(jnp.int32).reshape(b * s)
    table_rep = jnp.tile(table, (1, REP))
    out3 = _sc_lookup(b, s)(idx, table_rep)
    out5 = out3.reshape(s, EMBED_DIM // 8, b // 128, 8, 128)
    return out5.transpose(2, 4, 0, 1, 3).reshape(b, s, EMBED_DIM)
